# Initial kernel scaffold; baseline (speedup 1.0000x reference)
import jax
import jax.numpy as jnp
from jax.experimental import pallas as pl


def kernel(node_features, edge_index, agent_idx, W_rel1, b_rel1, W_root1, W_rel2, b_rel2, W_root2, Wp1, bp1, Wp2, bp2, Wp3, bp3):
    raise NotImplementedError("write your pallas kernel here")


# speedup vs baseline: 4.3811x; 4.3811x over previous
import functools

import jax
import jax.numpy as jnp
from jax import lax
from jax.experimental import pallas as pl
from jax.experimental.pallas import tpu as pltpu
from jax.experimental.pallas import tpu_sc as plsc

N = 10000
E = 320000
D = 128
A = 256
HOUT = 64

NC = 2
NS = 16
NW = NC * NS
BE = 128
NBLK = -(-E // (NW * BE))
EPW = NBLK * BE
EP = EPW * NW
NPAD = N + 112
RPT = NPAD // NS
APW = A // NW

_MESH = dict(core_axis_name="c", subcore_axis_name="s")


@functools.partial(
    pl.kernel,
    out_type=jax.ShapeDtypeStruct((NC, NPAD, D), jnp.float32),
    mesh=plsc.VectorSubcoreMesh(**_MESH),
    scratch_types=[
        pltpu.VMEM((NBLK, BE), jnp.int32),
        pltpu.VMEM((NBLK, BE), jnp.int32),
        pltpu.VMEM((BE, D), jnp.float32),
        pltpu.VMEM_SHARED((NPAD, D), jnp.float32),
        pltpu.SemaphoreType.DMA,
    ],
)
def _segsum_kernel(table, srcs, dsts, zeros, out, src_v, dst_v, rows_v, acc, sem):
    cid = lax.axis_index("c")
    sid = lax.axis_index("s")
    wid = sid * NC + cid
    pltpu.sync_copy(zeros, acc.at[pl.ds(sid * RPT, RPT)])
    plsc.subcore_barrier()
    pltpu.sync_copy(srcs.at[wid], src_v)
    pltpu.sync_copy(dsts.at[wid], dst_v)

    def body(j, carry):
        pltpu.async_copy(table.at[src_v.at[j]], rows_v, sem).wait()
        pltpu.sync_copy(rows_v, acc.at[dst_v.at[j]], add=True)
        return carry

    lax.fori_loop(0, NBLK, body, 0)
    plsc.subcore_barrier()
    pltpu.sync_copy(acc.at[pl.ds(sid * RPT, RPT)],
                    out.at[cid, pl.ds(sid * RPT, RPT)])


@functools.partial(
    pl.kernel,
    out_type=jax.ShapeDtypeStruct((3, A, D), jnp.float32),
    mesh=plsc.VectorSubcoreMesh(**_MESH),
    scratch_types=[
        pltpu.VMEM((APW,), jnp.int32),
        pltpu.VMEM((APW, D), jnp.float32),
        pltpu.SemaphoreType.DMA,
    ],
)
def _gather3_kernel(p0, p1, h, aidx, out, idx_v, rows_v, sem):
    cid = lax.axis_index("c")
    sid = lax.axis_index("s")
    wid = sid * NC + cid
    base = wid * APW
    pltpu.sync_copy(aidx.at[pl.ds(base, APW)], idx_v)
    pltpu.async_copy(p0.at[idx_v], rows_v, sem).wait()
    pltpu.sync_copy(rows_v, out.at[0, pl.ds(base, APW)])
    pltpu.async_copy(p1.at[idx_v], rows_v, sem).wait()
    pltpu.sync_copy(rows_v, out.at[1, pl.ds(base, APW)])
    pltpu.async_copy(h.at[idx_v], rows_v, sem).wait()
    pltpu.sync_copy(rows_v, out.at[2, pl.ds(base, APW)])


def _dot_t(a, w):
    return lax.dot_general(a, w, (((1,), (1,)), ((), ())),
                           preferred_element_type=jnp.float32)


RB = 1000


def _tc_linear(partials, x, w_rel, b_rel, w_root):
    def body(p_ref, x_ref, wr_ref, br_ref, wo_ref, o_ref):
        agg = p_ref[0] + p_ref[1]
        y = _dot_t(agg, wr_ref[...]) + br_ref[...] + _dot_t(x_ref[...], wo_ref[...])
        o_ref[...] = jnp.maximum(y, 0.0)

    return pl.pallas_call(
        body,
        grid=(N // RB,),
        in_specs=[
            pl.BlockSpec((2, RB, D), lambda i: (0, i, 0)),
            pl.BlockSpec((RB, D), lambda i: (i, 0)),
            pl.BlockSpec((D, D), lambda i: (0, 0)),
            pl.BlockSpec((1, D), lambda i: (0, 0)),
            pl.BlockSpec((D, D), lambda i: (0, 0)),
        ],
        out_specs=pl.BlockSpec((RB, D), lambda i: (i, 0)),
        out_shape=jax.ShapeDtypeStruct((N, D), jnp.float32),
    )(partials, x, w_rel, b_rel, w_root)


def _tc_head(sel3, w_rel2, b_rel2, w_root2, wp1, bp1, wp2, bp2, wp3, bp3):
    def body(s_ref, wr, br, wo, w1, b1, w2, b2, w3, b3, o_ref):
        agg = s_ref[0] + s_ref[1]
        emb = jnp.maximum(_dot_t(agg, wr[...]) + br[...] + _dot_t(s_ref[2], wo[...]), 0.0)
        t = jnp.maximum(_dot_t(emb, w1[...]) + b1[...], 0.0)
        t = jnp.maximum(_dot_t(t, w2[...]) + b2[...], 0.0)
        o_ref[...] = _dot_t(t, w3[...]) + b3[...]

    return pl.pallas_call(
        body,
        out_shape=jax.ShapeDtypeStruct((A, HOUT), jnp.float32),
    )(sel3, w_rel2, b_rel2, w_root2, wp1, bp1, wp2, bp2, wp3, bp3)


def kernel(node_features, edge_index, agent_idx,
           W_rel1, b_rel1, W_root1,
           W_rel2, b_rel2, W_root2,
           Wp1, bp1, Wp2, bp2, Wp3, bp3):
    pad = EP - E
    src3 = jnp.concatenate(
        [edge_index[0], jnp.zeros((pad,), jnp.int32)]).reshape(NW, NBLK, BE)
    dst3 = jnp.concatenate(
        [edge_index[1], jnp.full((pad,), N, jnp.int32)]).reshape(NW, NBLK, BE)
    zeros = jnp.zeros((RPT, D), jnp.float32)

    p1 = _segsum_kernel(node_features, src3, dst3, zeros)
    h = _tc_linear(p1[:, :N, :], node_features,
                   W_rel1, b_rel1.reshape(1, D), W_root1)
    p2 = _segsum_kernel(h, src3, dst3, zeros)
    sel3 = _gather3_kernel(p2[0], p2[1], h, agent_idx)
    out = _tc_head(sel3, W_rel2, b_rel2.reshape(1, D), W_root2,
                   Wp1, bp1.reshape(1, D), Wp2, bp2.reshape(1, D),
                   Wp3, bp3.reshape(1, HOUT))
    return out.reshape(A, 16, 4)
